# baseline (speedup 1.0000x reference)
import functools

import jax
import jax.numpy as jnp
from jax import lax
from jax.experimental import pallas as pl
from jax.experimental.pallas import tpu as pltpu
from jax.experimental.pallas import tpu_sc as plsc

_C = 64
_NE = 8192
_B = 16384
_BT = 256


def _mlp_blocks(h, Wt, b, g, bt, m, v):
    for i in range(Wt.shape[0]):
        h = jnp.dot(h, Wt[i], preferred_element_type=jnp.float32) + b[i]
        h = g[i] * (h - m[i]) / jnp.sqrt(v[i] + 1e-5) + bt[i]
        h = jnp.maximum(h, 0.0)
    return h


def _vq_head(h, W1t, b1, W2t, b2):
    t = jnp.tanh(jnp.dot(h, W1t, preferred_element_type=jnp.float32) + b1)
    return jnp.dot(t, W2t, preferred_element_type=jnp.float32) + b2


def _enc_kernel(x_ref, encWt_ref, encb_ref, encg_ref, encbt_ref, encm_ref,
                encv_ref, evq1t_ref, evqb1_ref, evq2t_ref, evqb2_ref, z_ref):
    h = _mlp_blocks(x_ref[...], encWt_ref, encb_ref, encg_ref, encbt_ref,
                    encm_ref, encv_ref)
    z_ref[...] = _vq_head(h, evq1t_ref[...], evqb1_ref[...], evq2t_ref[...],
                          evqb2_ref[...])


def _argmin_kernel(znT_ref, zn2b_ref, cbn_ref, c2col_ref, ind_ref):
    s = jnp.dot(cbn_ref[...], 2.0 * znT_ref[...],
                preferred_element_type=jnp.float32)
    dmat = (zn2b_ref[0:1, :] + c2col_ref[...]) - s
    mind = jnp.min(dmat, axis=0, keepdims=True)
    iota = lax.broadcasted_iota(jnp.int32, (_NE, _BT), 0)
    indrow = jnp.min(jnp.where(dmat == mind, iota, _NE), axis=0,
                     keepdims=True)
    ind_ref[...] = jnp.broadcast_to(indrow, (8, _BT))


def _dec_kernel(zn_ref, zqpair_ref, ind_ref, decWt_ref, decb_ref, decg_ref,
                decbt_ref, decm_ref, decv_ref, dvq1t_ref, dvqb1_ref,
                dvq2t_ref, dvqb2_ref, xrec_ref, loss_ref):
    zn = zn_ref[...]
    pair = zqpair_ref[...]
    parity = ind_ref[...] & 1
    z_q = jnp.where(parity == 0, pair[:, :_C], pair[:, _C:])
    diff = z_q - zn
    part = jnp.sum(diff * diff, keepdims=True)

    @pl.when(pl.program_id(0) == 0)
    def _():
        loss_ref[...] = jnp.zeros_like(loss_ref)

    loss_ref[...] += part

    zq_st = zn + (z_q - zn)
    h2 = _mlp_blocks(zq_st, decWt_ref, decb_ref, decg_ref, decbt_ref,
                     decm_ref, decv_ref)
    xrec_ref[...] = _vq_head(h2, dvq1t_ref[...], dvqb1_ref[...],
                             dvq2t_ref[...], dvqb2_ref[...])


def _full(shape):
    nd = len(shape)
    return pl.BlockSpec(shape, lambda i, _nd=nd: (0,) * _nd)


def _sc_gather(cbn_pairs, idx2d):
    info = plsc.get_sparse_core_info()
    nc, ns = info.num_cores, info.num_subcores
    nw = nc * ns
    bpw = _B // nw
    nch = bpw // 128
    mesh = plsc.VectorSubcoreMesh(core_axis_name="c", subcore_axis_name="s")

    @functools.partial(
        pl.kernel, mesh=mesh,
        out_type=jax.ShapeDtypeStruct((_B, 2 * _C), jnp.float32),
        scratch_types=[
            pltpu.VMEM((bpw,), jnp.int32),
            pltpu.VMEM((bpw, 2 * _C), jnp.float32),
            pltpu.SemaphoreType.DMA,
        ],
    )
    def gather(table_hbm, idx_hbm, out_hbm, idx_v, rows_v, sem):
        wid = lax.axis_index("s") * nc + lax.axis_index("c")
        pltpu.sync_copy(idx_hbm.at[wid], idx_v)
        handles = []
        for j in range(nch):
            handles.append(pltpu.async_copy(
                table_hbm.at[idx_v.at[pl.ds(j * 128, 128)]],
                rows_v.at[pl.ds(j * 128, 128)], sem))
        for h in handles:
            h.wait()
        pltpu.sync_copy(rows_v, out_hbm.at[pl.ds(wid * bpw, bpw)])

    return gather(cbn_pairs, idx2d)


def kernel(x, enc_W, enc_b, enc_gamma, enc_beta, enc_mean, enc_var,
           enc_vq_W1, enc_vq_b1, enc_vq_W2, enc_vq_b2,
           dec_W, dec_b, dec_gamma, dec_beta, dec_mean, dec_var,
           dec_vq_W1, dec_vq_b1, dec_vq_W2, dec_vq_b2, codebook):
    f32 = jnp.float32
    grid = (_B // _BT,)
    r2 = lambda a: a.reshape(a.shape[0], 1, _C)
    r1 = lambda a: a.reshape(1, _C)

    z = pl.pallas_call(
        _enc_kernel,
        grid=grid,
        in_specs=[
            pl.BlockSpec((_BT, _C), lambda i: (i, 0)),
            _full((2, _C, _C)), _full((2, 1, _C)), _full((2, 1, _C)),
            _full((2, 1, _C)), _full((2, 1, _C)), _full((2, 1, _C)),
            _full((_C, _C)), _full((1, _C)), _full((_C, _C)), _full((1, _C)),
        ],
        out_specs=pl.BlockSpec((_BT, _C), lambda i: (i, 0)),
        out_shape=jax.ShapeDtypeStruct((_B, _C), f32),
    )(x, jnp.transpose(enc_W, (0, 2, 1)), r2(enc_b), r2(enc_gamma),
      r2(enc_beta), r2(enc_mean), r2(enc_var),
      enc_vq_W1.T, r1(enc_vq_b1), enc_vq_W2.T, r1(enc_vq_b2))

    zn = lax.optimization_barrier(
        z / jnp.maximum(jnp.sqrt(jnp.sum(z * z, axis=-1, keepdims=True)),
                        1e-12))
    zn2 = lax.optimization_barrier(jnp.sum(zn * zn, axis=-1, keepdims=True))
    cbn = lax.optimization_barrier(codebook / jnp.maximum(
        jnp.sqrt(jnp.sum(codebook * codebook, axis=-1, keepdims=True)),
        1e-12))
    c2 = lax.optimization_barrier(jnp.sum(cbn * cbn, axis=-1))
    znT = lax.optimization_barrier(zn.T)
    zn2b = lax.optimization_barrier(
        jnp.broadcast_to(zn2.reshape(1, _B), (8, _B)))
    c2col = lax.optimization_barrier(c2.reshape(_NE, 1))

    ind8 = pl.pallas_call(
        _argmin_kernel,
        grid=grid,
        in_specs=[
            pl.BlockSpec((_C, _BT), lambda i: (0, i)),
            pl.BlockSpec((8, _BT), lambda i: (0, i)),
            _full((_NE, _C)), _full((_NE, 1)),
        ],
        out_specs=pl.BlockSpec((8, _BT), lambda i: (0, i)),
        out_shape=jax.ShapeDtypeStruct((8, _B), jnp.int32),
    )(znT, zn2b, cbn, c2col)
    ind = ind8[0].reshape(_B, 1)

    info = plsc.get_sparse_core_info()
    nw = info.num_cores * info.num_subcores
    zq_pair = _sc_gather(cbn.reshape(_NE // 2, 2 * _C),
                         (ind >> 1).reshape(nw, _B // nw))

    xrec, loss = pl.pallas_call(
        _dec_kernel,
        grid=grid,
        in_specs=[
            pl.BlockSpec((_BT, _C), lambda i: (i, 0)),
            pl.BlockSpec((_BT, 2 * _C), lambda i: (i, 0)),
            pl.BlockSpec((_BT, 1), lambda i: (i, 0)),
            _full((2, _C, _C)), _full((2, 1, _C)), _full((2, 1, _C)),
            _full((2, 1, _C)), _full((2, 1, _C)), _full((2, 1, _C)),
            _full((_C, _C)), _full((1, _C)), _full((_C, _C)), _full((1, _C)),
        ],
        out_specs=[
            pl.BlockSpec((_BT, _C), lambda i: (i, 0)),
            pl.BlockSpec((1, 1), lambda i: (0, 0)),
        ],
        out_shape=[jax.ShapeDtypeStruct((_B, _C), f32),
                   jax.ShapeDtypeStruct((1, 1), f32)],
    )(zn, zq_pair, ind, jnp.transpose(dec_W, (0, 2, 1)), r2(dec_b), r2(dec_gamma),
      r2(dec_beta), r2(dec_mean), r2(dec_var),
      dec_vq_W1.T, r1(dec_vq_b1), dec_vq_W2.T, r1(dec_vq_b2))

    emb_loss = loss[0, 0] / (_B * _C)
    return (xrec, emb_loss)
